# Initial kernel scaffold; baseline (speedup 1.0000x reference)
import jax
import jax.numpy as jnp
from jax.experimental import pallas as pl


def kernel(x, f_ij, idx_i, idx_j, rcut_ij, W_in, b_in, W_f1, b_f1, W_f2, b_f2, W_r1, b_r1, W_r2, b_r2):
    raise NotImplementedError("write your pallas kernel here")


# speedup vs baseline: 1.9104x; 1.9104x over previous
import functools

import jax
import jax.numpy as jnp
from jax import lax
from jax.experimental import pallas as pl
from jax.experimental.pallas import tpu as pltpu
from jax.experimental.pallas import tpu_sc as plsc

N_ATOMS = 10000
N_PAIRS = 320000
D = 128
N_RBF = 20

NC = 2
NS = 16
NW = NC * NS

PAD_ATOMS = 10240
ROWS_PER_TILE = PAD_ATOMS // NS
C = 80
PAIRS_PER_W = N_PAIRS // NW
NCHUNK = PAIRS_PER_W // C

_LOG2 = 0.6931471805599453


def _ssp(v):
    return jnp.maximum(v, 0.0) + jnp.log1p(jnp.exp(-jnp.abs(v))) - _LOG2


def _h_body(x_ref, w_ref, b_ref, o_ref):
    o_ref[...] = (
        jnp.dot(x_ref[...], w_ref[...], preferred_element_type=jnp.float32)
        + b_ref[...]
    )


def _filter_body(f_ref, rc_ref, w1_ref, b1_ref, w2_ref, b2_ref, o_ref):
    t = jnp.dot(f_ref[...], w1_ref[...], preferred_element_type=jnp.float32)
    t = _ssp(t + b1_ref[...])
    t = jnp.dot(t, w2_ref[...], preferred_element_type=jnp.float32) + b2_ref[...]
    o_ref[...] = t * rc_ref[...]


def _out_body(p0_ref, p1_ref, w1_ref, b1_ref, w2_ref, b2_ref, o_ref):
    agg = p0_ref[...] + p1_ref[...]
    t = jnp.dot(agg, w1_ref[...], preferred_element_type=jnp.float32)
    t = _ssp(t + b1_ref[...])
    o_ref[...] = (
        jnp.dot(t, w2_ref[...], preferred_element_type=jnp.float32) + b2_ref[...]
    )


_sc_mesh = plsc.VectorSubcoreMesh(core_axis_name="c", subcore_axis_name="s")


@functools.partial(
    pl.kernel,
    out_type=jax.ShapeDtypeStruct((NC, PAD_ATOMS, D), jnp.float32),
    mesh=_sc_mesh,
    scratch_types=[
        pltpu.VMEM((C,), jnp.int32),
        pltpu.VMEM((C,), jnp.int32),
        pltpu.VMEM((C, D), jnp.float32),
        pltpu.VMEM((C, D), jnp.float32),
        pltpu.VMEM_SHARED((PAD_ATOMS, D), jnp.float32),
        pltpu.SemaphoreType.DMA,
    ],
)
def _sc_gather_mul_scatter(h_hbm, idxj_hbm, idxi_hbm, wij_hbm, out_hbm,
                           idxj_v, idxi_v, rows_v, wij_v, agg_sh, sem):
    cid = lax.axis_index("c")
    sid = lax.axis_index("s")

    def zrow(r, carry):
        for cc in range(D // 16):
            rows_v[r, pl.ds(cc * 16, 16)] = jnp.zeros((16,), jnp.float32)
        return carry

    lax.fori_loop(0, C, zrow, 0)
    base_row = sid * ROWS_PER_TILE
    for z in range(ROWS_PER_TILE // C):
        pltpu.sync_copy(rows_v, agg_sh.at[pl.ds(base_row + z * C, C), :])
    plsc.subcore_barrier()

    pair_base = (cid * NS + sid) * PAIRS_PER_W

    def chunk(k, carry):
        b = pair_base + k * C
        pltpu.sync_copy(idxj_hbm.at[pl.ds(b, C)], idxj_v)
        pltpu.sync_copy(idxi_hbm.at[pl.ds(b, C)], idxi_v)
        pltpu.sync_copy(wij_hbm.at[pl.ds(b, C), :], wij_v)
        pltpu.async_copy(h_hbm.at[idxj_v], rows_v, sem).wait()

        def mrow(r, inner):
            for cc in range(D // 16):
                s = pl.ds(cc * 16, 16)
                rows_v[r, s] = rows_v[r, s] * wij_v[r, s]
            return inner

        lax.fori_loop(0, C, mrow, 0)
        pltpu.sync_copy(rows_v, agg_sh.at[idxi_v], add=True)
        return carry

    lax.fori_loop(0, NCHUNK, chunk, 0)
    plsc.subcore_barrier()

    pltpu.sync_copy(agg_sh.at[pl.ds(base_row, ROWS_PER_TILE), :],
                    out_hbm.at[cid, pl.ds(base_row, ROWS_PER_TILE), :])


def kernel(x, f_ij, idx_i, idx_j, rcut_ij,
           W_in, b_in, W_f1, b_f1, W_f2, b_f2, W_r1, b_r1, W_r2, b_r2):
    x2 = x.reshape(N_ATOMS, D)

    h = pl.pallas_call(
        _h_body,
        out_shape=jax.ShapeDtypeStruct((N_ATOMS, D), jnp.float32),
    )(x2, W_in, b_in.reshape(1, D))

    pb = 2560
    wij = pl.pallas_call(
        _filter_body,
        grid=(N_PAIRS // pb,),
        in_specs=[
            pl.BlockSpec((pb, N_RBF), lambda i: (i, 0)),
            pl.BlockSpec((pb, 1), lambda i: (i, 0)),
            pl.BlockSpec((N_RBF, D), lambda i: (0, 0)),
            pl.BlockSpec((1, D), lambda i: (0, 0)),
            pl.BlockSpec((D, D), lambda i: (0, 0)),
            pl.BlockSpec((1, D), lambda i: (0, 0)),
        ],
        out_specs=pl.BlockSpec((pb, D), lambda i: (i, 0)),
        out_shape=jax.ShapeDtypeStruct((N_PAIRS, D), jnp.float32),
    )(f_ij, rcut_ij.reshape(N_PAIRS, 1), W_r1, b_r1.reshape(1, D),
      W_r2, b_r2.reshape(1, D))

    partials = _sc_gather_mul_scatter(
        h, idx_j.astype(jnp.int32), idx_i.astype(jnp.int32), wij)

    out = pl.pallas_call(
        _out_body,
        out_shape=jax.ShapeDtypeStruct((N_ATOMS, D), jnp.float32),
    )(partials[0, :N_ATOMS], partials[1, :N_ATOMS],
      W_f1, b_f1.reshape(1, D), W_f2, b_f2.reshape(1, D))

    return out.reshape(1, N_ATOMS, D)
